# baseline (speedup 1.0000x reference)
import functools

import jax
import jax.numpy as jnp
import numpy as np
from jax import lax
from jax.experimental import pallas as pl
from jax.experimental.pallas import tpu as pltpu
from jax.experimental.pallas import tpu_sc as plsc

MIN_NORM = 1e-15
EPS = 4e-3

N_NODES = 10000
N_EDGES = 160000
D = 256
HALF = 128

NS = 16
CHUNK = 128
CHUNKS_PER_TILE = 80
E_PAD = NS * CHUNKS_PER_TILE * CHUNK
N_PAD = 10240
ROWS_PER_TILE = N_PAD // NS
ROW_CHUNK = 128
ROW_CHUNKS = ROWS_PER_TILE // ROW_CHUNK


def _rsum(v):
    ones = jnp.ones((v.shape[-1], 1), jnp.float32)
    return lax.dot_general(v, ones, (((1,), (0,)), ((), ())),
                           precision=lax.Precision.HIGHEST,
                           preferred_element_type=jnp.float32)


def _norm(x):
    return jnp.clip(jnp.sqrt(_rsum(x * x)), MIN_NORM, None)


def _artanh(x):
    x = jnp.clip(x, -1.0 + 1e-7, 1.0 - 1e-7)
    return 0.5 * jnp.log((1.0 + x) / (1.0 - x))


def _proj(x):
    norm = _norm(x)
    maxnorm = 1.0 - EPS
    return jnp.where(norm > maxnorm, x / norm * maxnorm, x)


def _expmap0(u):
    u_norm = _norm(u)
    return jnp.tanh(u_norm) * u / u_norm


def _logmap0(p):
    p_norm = _norm(p)
    return _artanh(p_norm) * p / p_norm


def _projexp(u):
    n = _norm(u)
    return u * (jnp.minimum(jnp.tanh(n), 1.0 - EPS) / n)


def _projexp_n(u):
    n = _norm(u)
    s = jnp.minimum(jnp.tanh(n), 1.0 - EPS)
    return u * (s / n), s


_ARTANH_MAX = float(
    np.float32(0.5) * np.log((np.float32(1) + np.float32(1.0 - EPS))
                             / (np.float32(1) - np.float32(1.0 - EPS))))


def _mobius_add(x, y):
    x2 = _rsum(x * x)
    y2 = _rsum(y * y)
    xy = _rsum(x * y)
    num = (1.0 + 2.0 * xy + y2) * x + (1.0 - x2) * y
    denom = 1.0 + 2.0 * xy + x2 * y2
    return num / jnp.clip(denom, MIN_NORM, None)


def _hyp_linear(xh, x_norm, w, b_row):
    mx = lax.dot_general(xh, w, (((1,), (1,)), ((), ())),
                         precision=lax.Precision.DEFAULT,
                         preferred_element_type=jnp.float32)
    mx_norm = _norm(mx)
    scale = jnp.minimum(jnp.tanh(mx_norm / x_norm * _artanh(x_norm)),
                        1.0 - EPS) / mx_norm
    res = mx * scale
    hyp_b = _projexp(b_row)
    return _proj(_mobius_add(res, hyp_b))


def _post_agg(agg):
    n = _norm(agg)
    xt = jax.nn.relu(agg * (jnp.minimum(n, _ARTANH_MAX) / n))
    return _projexp_n(xt)


def _pre1_body(x_ref, w_ref, b_ref, lo_ref, hi_ref):
    x = x_ref[...]
    xh, xn = _projexp_n(x)
    xt = _logmap0(_hyp_linear(xh, xn, w_ref[...], b_ref[...]))
    lo_ref[...] = xt[:, :HALF]
    hi_ref[...] = xt[:, HALF:]


def _mid_body(lo_in, hi_in, w_ref, b_ref, lo_ref, hi_ref):
    agg = jnp.concatenate([lo_in[...], hi_in[...]], axis=1)
    u, un = _post_agg(agg)
    xt = _logmap0(_hyp_linear(u, un, w_ref[...], b_ref[...]))
    lo_ref[...] = xt[:, :HALF]
    hi_ref[...] = xt[:, HALF:]


def _final_body(lo_in, hi_in, out_ref):
    agg = jnp.concatenate([lo_in[...], hi_in[...]], axis=1)
    out_ref[...] = _post_agg(agg)[0]


_BLK = 2000
_GRID = N_NODES // _BLK

_row_spec = pl.BlockSpec((_BLK, D), lambda i: (i, 0))
_half_spec = pl.BlockSpec((_BLK, HALF), lambda i: (i, 0))
_w_spec = pl.BlockSpec((D, D), lambda i: (0, 0))
_b_spec = pl.BlockSpec((1, D), lambda i: (0, 0))

_half_sds = jax.ShapeDtypeStruct((N_NODES, HALF), jnp.float32)

_pre1 = pl.pallas_call(
    _pre1_body,
    grid=(_GRID,),
    in_specs=[_row_spec, _w_spec, _b_spec],
    out_specs=[_half_spec, _half_spec],
    out_shape=[_half_sds, _half_sds],
)

_mid = pl.pallas_call(
    _mid_body,
    grid=(_GRID,),
    in_specs=[_half_spec, _half_spec, _w_spec, _b_spec],
    out_specs=[_half_spec, _half_spec],
    out_shape=[_half_sds, _half_sds],
)

_final = pl.pallas_call(
    _final_body,
    grid=(_GRID,),
    in_specs=[_half_spec, _half_spec],
    out_specs=_row_spec,
    out_shape=jax.ShapeDtypeStruct((N_NODES, D), jnp.float32),
)


def _seg_sum_body(lo_hbm, hi_hbm, idx_hbm, out_lo, out_hi,
                  accum, ring, rows0, rows1,
                  gi0, gi1, gi2, gi3, gr0, gr1):
    c = lax.axis_index("c")
    s = lax.axis_index("s")
    gis = [gi0, gi1, gi2, gi3]
    kbase = s * CHUNKS_PER_TILE

    def idx_load(k, slot, sem):
        pltpu.async_copy(idx_hbm.at[kbase + k], ring.at[slot], sem)

    def idx_wait(slot, sem):
        pltpu.make_async_copy(idx_hbm.at[0], ring.at[slot], sem).wait()

    def gather(table, k_slot, buf, sem):
        pltpu.async_copy(table.at[ring.at[k_slot, 0]], buf, sem)

    def gather_wait(table, buf, sem):
        pltpu.make_async_copy(table.at[pl.ds(0, CHUNK)], buf, sem).wait()

    def scatter(slot, buf):
        pltpu.sync_copy(buf, accum.at[ring.at[slot, 1]], add=True)

    for b in range(4):
        idx_load(b, b, gis[b])

    zero = jnp.zeros((16,), jnp.float32)

    def zrow(i, _):
        def zcol(j, _):
            rows1[i, pl.ds(j * 16, 16)] = zero
            return 0
        return lax.fori_loop(0, HALF // 16, zcol, 0)

    lax.fori_loop(0, ROW_CHUNK, zrow, 0)
    zcopies = [
        pltpu.async_copy(
            rows1, accum.at[pl.ds(s * ROWS_PER_TILE + q * ROW_CHUNK,
                                  ROW_CHUNK)], gr1)
        for q in range(ROW_CHUNKS)]
    for h in zcopies:
        h.wait()

    plsc.subcore_barrier()

    def run(table, out_hbm):
        idx_wait(0, gi0)
        gather(table, 0, rows0, gr0)
        idx_wait(1, gi1)
        gather(table, 1, rows1, gr1)

        def step(t, _):
            c0 = 4 * t

            gather_wait(table, rows0, gr0)
            scatter(0, rows0)
            idx_wait(2, gi2)
            gather(table, 2, rows0, gr0)
            @pl.when(c0 + 4 < CHUNKS_PER_TILE)
            def _():
                idx_load(c0 + 4, 0, gi0)

            gather_wait(table, rows1, gr1)
            scatter(1, rows1)
            idx_wait(3, gi3)
            gather(table, 3, rows1, gr1)
            @pl.when(c0 + 5 < CHUNKS_PER_TILE)
            def _():
                idx_load(c0 + 5, 1, gi1)

            gather_wait(table, rows0, gr0)
            scatter(2, rows0)
            @pl.when(c0 + 4 < CHUNKS_PER_TILE)
            def _():
                idx_wait(0, gi0)
                gather(table, 0, rows0, gr0)
            @pl.when(c0 + 6 < CHUNKS_PER_TILE)
            def _():
                idx_load(c0 + 6, 2, gi2)

            gather_wait(table, rows1, gr1)
            scatter(3, rows1)
            @pl.when(c0 + 5 < CHUNKS_PER_TILE)
            def _():
                idx_wait(1, gi1)
                gather(table, 1, rows1, gr1)
            @pl.when(c0 + 7 < CHUNKS_PER_TILE)
            def _():
                idx_load(c0 + 7, 3, gi3)
            return 0

        lax.fori_loop(0, CHUNKS_PER_TILE // 4, step, 0)
        plsc.subcore_barrier()

        def obase(q):
            return s * ROWS_PER_TILE + q * ROW_CHUNK

        bufs = [rows0, rows1]
        sems = [gr0, gr1]
        pltpu.async_copy(accum.at[pl.ds(obase(0), ROW_CHUNK)], rows0, gr0)
        for q in range(ROW_CHUNKS):
            b = q % 2
            pltpu.make_async_copy(
                accum.at[pl.ds(obase(q), ROW_CHUNK)], bufs[b],
                sems[b]).wait()
            if q + 1 < ROW_CHUNKS:
                pltpu.async_copy(
                    accum.at[pl.ds(obase(q + 1), ROW_CHUNK)],
                    bufs[(q + 1) % 2], sems[(q + 1) % 2])
            pltpu.sync_copy(bufs[b], out_hbm.at[pl.ds(obase(q), ROW_CHUNK)])

    @pl.when(c == 0)
    def _():
        run(lo_hbm, out_lo)

    @pl.when(c == 1)
    def _():
        run(hi_hbm, out_hi)


_pad_sds = jax.ShapeDtypeStruct((N_PAD, HALF), jnp.float32)


@functools.cache
def _get_seg_sum():
    return functools.partial(
        pl.kernel,
        out_type=[_pad_sds, _pad_sds],
        mesh=plsc.VectorSubcoreMesh(core_axis_name="c",
                                    subcore_axis_name="s"),
        scratch_types=[
            pltpu.VMEM_SHARED((N_PAD, HALF), jnp.float32),
            pltpu.VMEM((4, 2, CHUNK), jnp.int32),
            pltpu.VMEM((CHUNK, HALF), jnp.float32),
            pltpu.VMEM((CHUNK, HALF), jnp.float32),
            pltpu.SemaphoreType.DMA,
            pltpu.SemaphoreType.DMA,
            pltpu.SemaphoreType.DMA,
            pltpu.SemaphoreType.DMA,
            pltpu.SemaphoreType.DMA,
            pltpu.SemaphoreType.DMA,
        ],
    )(_seg_sum_body)


def kernel(x, edge_index, W1, b1, W2, b2):
    n_extra = E_PAD - N_EDGES
    src_pad = jnp.arange(n_extra, dtype=jnp.int32) % N_NODES
    dst_pad = N_NODES + (jnp.arange(n_extra, dtype=jnp.int32)
                         % (N_PAD - N_NODES))
    src = jnp.concatenate(
        [edge_index[0].astype(jnp.int32), src_pad]).reshape(
            NS * CHUNKS_PER_TILE, 1, CHUNK)
    dst = jnp.concatenate(
        [edge_index[1].astype(jnp.int32), dst_pad]).reshape(
            NS * CHUNKS_PER_TILE, 1, CHUNK)
    idx = jnp.concatenate([src, dst], axis=1)

    seg_sum = _get_seg_sum()
    lo1, hi1 = _pre1(x, W1, b1.reshape(1, D))
    alo1, ahi1 = seg_sum(lo1, hi1, idx)
    lo2, hi2 = _mid(alo1, ahi1, W2, b2.reshape(1, D))
    alo2, ahi2 = seg_sum(lo2, hi2, idx)
    return _final(alo2, ahi2)


# speedup vs baseline: 1.6006x; 1.6006x over previous
import functools

import jax
import jax.numpy as jnp
import numpy as np
from jax import lax
from jax.experimental import pallas as pl
from jax.experimental.pallas import tpu as pltpu
from jax.experimental.pallas import tpu_sc as plsc

MIN_NORM = 1e-15
EPS = 4e-3

N_NODES = 10000
N_EDGES = 160000
D = 256
HALF = 128

NS = 16
CHUNK = 128
CHUNKS_PER_TILE = 80
E_PAD = NS * CHUNKS_PER_TILE * CHUNK
N_PAD = 10240
ROWS_PER_TILE = N_PAD // NS
ROW_CHUNK = 128
ROW_CHUNKS = ROWS_PER_TILE // ROW_CHUNK


def _rsum(v):
    return jnp.sum(v, axis=-1, keepdims=True)


def _norm(x):
    return jnp.clip(jnp.sqrt(_rsum(x * x)), MIN_NORM, None)


def _artanh(x):
    x = jnp.clip(x, -1.0 + 1e-7, 1.0 - 1e-7)
    return 0.5 * jnp.log((1.0 + x) / (1.0 - x))


def _proj(x):
    norm = _norm(x)
    maxnorm = 1.0 - EPS
    return jnp.where(norm > maxnorm, x / norm * maxnorm, x)


def _expmap0(u):
    u_norm = _norm(u)
    return jnp.tanh(u_norm) * u / u_norm


def _logmap0(p):
    p_norm = _norm(p)
    return _artanh(p_norm) * p / p_norm


def _projexp(u):
    n = _norm(u)
    return u * (jnp.minimum(jnp.tanh(n), 1.0 - EPS) / n)


def _projexp_n(u):
    n = _norm(u)
    s = jnp.minimum(jnp.tanh(n), 1.0 - EPS)
    return u * (s / n), s


_ARTANH_MAX = float(
    np.float32(0.5) * np.log((np.float32(1) + np.float32(1.0 - EPS))
                             / (np.float32(1) - np.float32(1.0 - EPS))))


def _mobius_add(x, y):
    x2 = _rsum(x * x)
    y2 = _rsum(y * y)
    xy = _rsum(x * y)
    num = (1.0 + 2.0 * xy + y2) * x + (1.0 - x2) * y
    denom = 1.0 + 2.0 * xy + x2 * y2
    return num / jnp.clip(denom, MIN_NORM, None)


def _hyp_linear(xh, x_norm, w, b_row):
    mx = lax.dot_general(xh, w, (((1,), (1,)), ((), ())),
                         precision=lax.Precision.DEFAULT,
                         preferred_element_type=jnp.float32)
    mx_norm = _norm(mx)
    scale = jnp.minimum(jnp.tanh(mx_norm / x_norm * _artanh(x_norm)),
                        1.0 - EPS) / mx_norm
    res = mx * scale
    hyp_b = _projexp(b_row)
    return _proj(_mobius_add(res, hyp_b))


def _post_agg(agg):
    n = _norm(agg)
    xt = jax.nn.relu(agg * (jnp.minimum(n, _ARTANH_MAX) / n))
    return _projexp_n(xt)


def _pre1_body(x_ref, w_ref, b_ref, lo_ref, hi_ref):
    x = x_ref[...]
    xh, xn = _projexp_n(x)
    xt = _logmap0(_hyp_linear(xh, xn, w_ref[...], b_ref[...]))
    lo_ref[...] = xt[:, :HALF]
    hi_ref[...] = xt[:, HALF:]


def _mid_body(lo_in, hi_in, w_ref, b_ref, lo_ref, hi_ref):
    agg = jnp.concatenate([lo_in[...], hi_in[...]], axis=1)
    u, un = _post_agg(agg)
    xt = _logmap0(_hyp_linear(u, un, w_ref[...], b_ref[...]))
    lo_ref[...] = xt[:, :HALF]
    hi_ref[...] = xt[:, HALF:]


def _final_body(lo_in, hi_in, out_ref):
    agg = jnp.concatenate([lo_in[...], hi_in[...]], axis=1)
    out_ref[...] = _post_agg(agg)[0]


_BLK = 2000
_GRID = N_NODES // _BLK

_row_spec = pl.BlockSpec((_BLK, D), lambda i: (i, 0))
_half_spec = pl.BlockSpec((_BLK, HALF), lambda i: (i, 0))
_w_spec = pl.BlockSpec((D, D), lambda i: (0, 0))
_b_spec = pl.BlockSpec((1, D), lambda i: (0, 0))

_half_sds = jax.ShapeDtypeStruct((N_NODES, HALF), jnp.float32)

_pre1 = pl.pallas_call(
    _pre1_body,
    grid=(_GRID,),
    in_specs=[_row_spec, _w_spec, _b_spec],
    out_specs=[_half_spec, _half_spec],
    out_shape=[_half_sds, _half_sds],
)

_mid = pl.pallas_call(
    _mid_body,
    grid=(_GRID,),
    in_specs=[_half_spec, _half_spec, _w_spec, _b_spec],
    out_specs=[_half_spec, _half_spec],
    out_shape=[_half_sds, _half_sds],
)

_final = pl.pallas_call(
    _final_body,
    grid=(_GRID,),
    in_specs=[_half_spec, _half_spec],
    out_specs=_row_spec,
    out_shape=jax.ShapeDtypeStruct((N_NODES, D), jnp.float32),
)


def _seg_sum_body(lo_hbm, hi_hbm, idx_hbm, out_lo, out_hi,
                  accum, ring, rows0, rows1,
                  gi0, gi1, gi2, gi3, gr0, gr1):
    c = lax.axis_index("c")
    s = lax.axis_index("s")
    gis = [gi0, gi1, gi2, gi3]
    kbase = s * CHUNKS_PER_TILE

    def idx_load(k, slot, sem):
        pltpu.async_copy(idx_hbm.at[kbase + k], ring.at[slot], sem)

    def idx_wait(slot, sem):
        pltpu.make_async_copy(idx_hbm.at[0], ring.at[slot], sem).wait()

    def gather(table, k_slot, buf, sem):
        pltpu.async_copy(table.at[ring.at[k_slot, 0]], buf, sem)

    def gather_wait(table, buf, sem):
        pltpu.make_async_copy(table.at[pl.ds(0, CHUNK)], buf, sem).wait()

    def scatter(slot, buf):
        pltpu.sync_copy(buf, accum.at[ring.at[slot, 1]], add=True)

    for b in range(4):
        idx_load(b, b, gis[b])

    zero = jnp.zeros((16,), jnp.float32)

    def zrow(i, _):
        def zcol(j, _):
            rows1[i, pl.ds(j * 16, 16)] = zero
            return 0
        return lax.fori_loop(0, HALF // 16, zcol, 0)

    lax.fori_loop(0, ROW_CHUNK, zrow, 0)
    zcopies = [
        pltpu.async_copy(
            rows1, accum.at[pl.ds(s * ROWS_PER_TILE + q * ROW_CHUNK,
                                  ROW_CHUNK)], gr1)
        for q in range(ROW_CHUNKS)]
    for h in zcopies:
        h.wait()

    plsc.subcore_barrier()

    def run(table, out_hbm):
        idx_wait(0, gi0)
        gather(table, 0, rows0, gr0)
        idx_wait(1, gi1)
        gather(table, 1, rows1, gr1)

        def step(t, _):
            c0 = 4 * t

            gather_wait(table, rows0, gr0)
            scatter(0, rows0)
            idx_wait(2, gi2)
            gather(table, 2, rows0, gr0)
            @pl.when(c0 + 4 < CHUNKS_PER_TILE)
            def _():
                idx_load(c0 + 4, 0, gi0)

            gather_wait(table, rows1, gr1)
            scatter(1, rows1)
            idx_wait(3, gi3)
            gather(table, 3, rows1, gr1)
            @pl.when(c0 + 5 < CHUNKS_PER_TILE)
            def _():
                idx_load(c0 + 5, 1, gi1)

            gather_wait(table, rows0, gr0)
            scatter(2, rows0)
            @pl.when(c0 + 4 < CHUNKS_PER_TILE)
            def _():
                idx_wait(0, gi0)
                gather(table, 0, rows0, gr0)
            @pl.when(c0 + 6 < CHUNKS_PER_TILE)
            def _():
                idx_load(c0 + 6, 2, gi2)

            gather_wait(table, rows1, gr1)
            scatter(3, rows1)
            @pl.when(c0 + 5 < CHUNKS_PER_TILE)
            def _():
                idx_wait(1, gi1)
                gather(table, 1, rows1, gr1)
            @pl.when(c0 + 7 < CHUNKS_PER_TILE)
            def _():
                idx_load(c0 + 7, 3, gi3)
            return 0

        lax.fori_loop(0, CHUNKS_PER_TILE // 4, step, 0)
        plsc.subcore_barrier()

        def obase(q):
            return s * ROWS_PER_TILE + q * ROW_CHUNK

        bufs = [rows0, rows1]
        sems = [gr0, gr1]
        pltpu.async_copy(accum.at[pl.ds(obase(0), ROW_CHUNK)], rows0, gr0)
        for q in range(ROW_CHUNKS):
            b = q % 2
            pltpu.make_async_copy(
                accum.at[pl.ds(obase(q), ROW_CHUNK)], bufs[b],
                sems[b]).wait()
            if q + 1 < ROW_CHUNKS:
                pltpu.async_copy(
                    accum.at[pl.ds(obase(q + 1), ROW_CHUNK)],
                    bufs[(q + 1) % 2], sems[(q + 1) % 2])
            pltpu.sync_copy(bufs[b], out_hbm.at[pl.ds(obase(q), ROW_CHUNK)])

    @pl.when(c == 0)
    def _():
        run(lo_hbm, out_lo)

    @pl.when(c == 1)
    def _():
        run(hi_hbm, out_hi)


_pad_sds = jax.ShapeDtypeStruct((N_PAD, HALF), jnp.float32)


@functools.cache
def _get_seg_sum():
    return functools.partial(
        pl.kernel,
        out_type=[_pad_sds, _pad_sds],
        mesh=plsc.VectorSubcoreMesh(core_axis_name="c",
                                    subcore_axis_name="s"),
        scratch_types=[
            pltpu.VMEM_SHARED((N_PAD, HALF), jnp.float32),
            pltpu.VMEM((4, 2, CHUNK), jnp.int32),
            pltpu.VMEM((CHUNK, HALF), jnp.float32),
            pltpu.VMEM((CHUNK, HALF), jnp.float32),
            pltpu.SemaphoreType.DMA,
            pltpu.SemaphoreType.DMA,
            pltpu.SemaphoreType.DMA,
            pltpu.SemaphoreType.DMA,
            pltpu.SemaphoreType.DMA,
            pltpu.SemaphoreType.DMA,
        ],
    )(_seg_sum_body)


def kernel(x, edge_index, W1, b1, W2, b2):
    n_extra = E_PAD - N_EDGES
    src_pad = jnp.arange(n_extra, dtype=jnp.int32) % N_NODES
    dst_pad = N_NODES + (jnp.arange(n_extra, dtype=jnp.int32)
                         % (N_PAD - N_NODES))
    src = jnp.concatenate(
        [edge_index[0].astype(jnp.int32), src_pad]).reshape(
            NS * CHUNKS_PER_TILE, 1, CHUNK)
    dst = jnp.concatenate(
        [edge_index[1].astype(jnp.int32), dst_pad]).reshape(
            NS * CHUNKS_PER_TILE, 1, CHUNK)
    idx = jnp.concatenate([src, dst], axis=1)

    seg_sum = _get_seg_sum()
    lo1, hi1 = _pre1(x, W1, b1.reshape(1, D))
    alo1, ahi1 = seg_sum(lo1, hi1, idx)
    lo2, hi2 = _mid(alo1, ahi1, W2, b2.reshape(1, D))
    alo2, ahi2 = seg_sum(lo2, hi2, idx)
    return _final(alo2, ahi2)
